# baseline (speedup 1.0000x reference)
import functools

import jax
import jax.numpy as jnp
from jax import lax
from jax.experimental import pallas as pl
from jax.experimental.pallas import tpu as pltpu
from jax.experimental.pallas import tpu_sc as plsc

_ANCHORS = ((10.0, 13.0), (16.0, 30.0), (33.0, 23.0))
_G = 76
_CELLS = _G * _G
_NA = 3
_ATTRS = 85
_CH = _NA * _ATTRS
_B = 32
_STRIDE = 8.0
_LANES = 16
_WSTARTS = (0, 16, 32, 48, 60)
_AP = 88


def _transform_row(in_v, out_v, r):
    iota = lax.iota(jnp.int32, _LANES)

    for a in range(_NA):
        @plsc.parallel_loop(0, _ATTRS, unroll=5)
        def _sigmoid_all(t):
            tb = jnp.zeros((_LANES,), jnp.int32) + t
            for w0 in _WSTARTS:
                rows = (iota + w0) * _NA + a
                v = in_v[a * _ATTRS + t, pl.ds(w0, _LANES)]
                s = 1.0 / (1.0 + jnp.exp(-v))
                plsc.store_scatter(out_v, [rows, tb], s)

    cy8 = r.astype(jnp.float32) * _STRIDE
    zeros = jnp.zeros((_LANES,), jnp.int32)
    for w0 in _WSTARTS:
        cx8 = (iota + w0).astype(jnp.float32) * _STRIDE
        for a in range(_NA):
            base = a * _ATTRS
            rows = (iota + w0) * _NA + a
            vx = in_v[base + 0, pl.ds(w0, _LANES)]
            sx = 1.0 / (1.0 + jnp.exp(-vx))
            plsc.store_scatter(out_v, [rows, zeros], sx * _STRIDE + cx8)
            vy = in_v[base + 1, pl.ds(w0, _LANES)]
            sy = 1.0 / (1.0 + jnp.exp(-vy))
            plsc.store_scatter(out_v, [rows, zeros + 1], sy * _STRIDE + cy8)
            vw = in_v[base + 2, pl.ds(w0, _LANES)]
            plsc.store_scatter(out_v, [rows, zeros + 2],
                               jnp.exp(vw) * _ANCHORS[a][0])
            vh = in_v[base + 3, pl.ds(w0, _LANES)]
            plsc.store_scatter(out_v, [rows, zeros + 3],
                               jnp.exp(vh) * _ANCHORS[a][1])


def _make_sc_kernel():
    mesh = plsc.VectorSubcoreMesh(core_axis_name="c", subcore_axis_name="s")

    @functools.partial(
        pl.kernel,
        out_type=jax.ShapeDtypeStruct((_B, _CELLS * _NA, 128), jnp.float32),
        mesh=mesh,
        scratch_types=[
            pltpu.VMEM((_CH, _G), jnp.float32),
            pltpu.VMEM((_CH, _G), jnp.float32),
            pltpu.VMEM((_G * _NA, _AP), jnp.float32),
            pltpu.VMEM((_G * _NA, _AP), jnp.float32),
            pltpu.SemaphoreType.DMA,
            pltpu.SemaphoreType.DMA,
            pltpu.SemaphoreType.DMA,
            pltpu.SemaphoreType.DMA,
        ],
        compiler_params=pltpu.CompilerParams(
            use_tc_tiling_on_sc=False, needs_layout_passes=False
        ),
    )
    def yolo_sc(x_hbm, out_hbm, in0, in1, o0, o1, si0, si1, so0, so1):
        b = lax.axis_index("s") * 2 + lax.axis_index("c")
        ins = (in0, in1)
        outs = (o0, o1)
        sis = (si0, si1)
        sos = (so0, so1)
        def in_copy(r, i):
            return pltpu.make_async_copy(
                x_hbm.at[b, :, r, :], ins[i], sis[i]
            )

        def out_copy(r, i):
            return pltpu.make_async_copy(
                outs[i],
                out_hbm.at[b, pl.ds(r * _G * _NA, _G * _NA), pl.ds(0, _AP)],
                sos[i],
            )

        in_copy(0, 0).start()
        in_copy(1, 1).start()

        @pl.loop(0, _G, step=2)
        def _rows(r):
            for i in range(2):
                rr = r + i
                in_copy(rr, i).wait()

                @pl.when(rr >= 2)
                def _():
                    out_copy(rr - 2, i).wait()

                _transform_row(ins[i], outs[i], rr)
                out_copy(rr, i).start()

                @pl.when(rr + 2 < _G)
                def _():
                    in_copy(rr + 2, i).start()

        out_copy(_G - 2, 0).wait()
        out_copy(_G - 1, 1).wait()

    return yolo_sc


_yolo_sc = _make_sc_kernel()


@jax.jit
def kernel(x):
    y_p = _yolo_sc(x)
    return y_p[:, :, :_ATTRS]


# speedup vs baseline: 1.6289x; 1.6289x over previous
import functools

import jax
import jax.numpy as jnp
from jax import lax
from jax.experimental import pallas as pl
from jax.experimental.pallas import tpu as pltpu
from jax.experimental.pallas import tpu_sc as plsc

_ANCHORS = ((10.0, 13.0), (16.0, 30.0), (33.0, 23.0))
_G = 76
_CELLS = _G * _G
_NA = 3
_ATTRS = 85
_CH = _NA * _ATTRS
_B = 32
_STRIDE = 8.0
_LANES = 16
_WSTARTS = (0, 16, 32, 48, 64)
_GP = 80
_AP = 88


def _transform_row(in_v, out_v, r):
    iota = lax.iota(jnp.int32, _LANES)

    in_mask = iota < (_G - _WSTARTS[-1])

    for a in range(_NA):
        @plsc.parallel_loop(0, _ATTRS, unroll=5)
        def _sigmoid_all(t):
            tb = jnp.zeros((_LANES,), jnp.int32) + t
            for w0 in _WSTARTS:
                rows = (iota + w0) * _NA + a
                v = in_v[a * _ATTRS + t, pl.ds(w0, _LANES)]
                s = 1.0 / (1.0 + jnp.exp(-v))
                m = in_mask if w0 == _WSTARTS[-1] else None
                plsc.store_scatter(out_v, [rows, tb], s, mask=m)

    cy8 = r.astype(jnp.float32) * _STRIDE
    zeros = jnp.zeros((_LANES,), jnp.int32)
    for w0 in _WSTARTS:
        cx8 = (iota + w0).astype(jnp.float32) * _STRIDE
        m = in_mask if w0 == _WSTARTS[-1] else None
        for a in range(_NA):
            base = a * _ATTRS
            rows = (iota + w0) * _NA + a
            vx = in_v[base + 0, pl.ds(w0, _LANES)]
            sx = 1.0 / (1.0 + jnp.exp(-vx))
            plsc.store_scatter(out_v, [rows, zeros], sx * _STRIDE + cx8,
                               mask=m)
            vy = in_v[base + 1, pl.ds(w0, _LANES)]
            sy = 1.0 / (1.0 + jnp.exp(-vy))
            plsc.store_scatter(out_v, [rows, zeros + 1], sy * _STRIDE + cy8,
                               mask=m)
            vw = in_v[base + 2, pl.ds(w0, _LANES)]
            plsc.store_scatter(out_v, [rows, zeros + 2],
                               jnp.exp(vw) * _ANCHORS[a][0], mask=m)
            vh = in_v[base + 3, pl.ds(w0, _LANES)]
            plsc.store_scatter(out_v, [rows, zeros + 3],
                               jnp.exp(vh) * _ANCHORS[a][1], mask=m)


def _make_sc_kernel():
    mesh = plsc.VectorSubcoreMesh(core_axis_name="c", subcore_axis_name="s")

    @functools.partial(
        pl.kernel,
        out_type=jax.ShapeDtypeStruct((_B, _CELLS * _NA, 128), jnp.float32),
        mesh=mesh,
        scratch_types=[
            pltpu.VMEM((_CH, _GP), jnp.float32),
            pltpu.VMEM((_CH, _GP), jnp.float32),
            pltpu.VMEM((_G * _NA, _AP), jnp.float32),
            pltpu.VMEM((_G * _NA, _AP), jnp.float32),
            pltpu.SemaphoreType.DMA,
            pltpu.SemaphoreType.DMA,
            pltpu.SemaphoreType.DMA,
            pltpu.SemaphoreType.DMA,
        ],
        compiler_params=pltpu.CompilerParams(
            use_tc_tiling_on_sc=False, needs_layout_passes=False
        ),
    )
    def yolo_sc(x_hbm, out_hbm, in0, in1, o0, o1, si0, si1, so0, so1):
        b = lax.axis_index("s") * 2 + lax.axis_index("c")
        ins = (in0, in1)
        outs = (o0, o1)
        sis = (si0, si1)
        sos = (so0, so1)
        def in_copy(r, i):
            return pltpu.make_async_copy(
                x_hbm.at[b, :, r, pl.ds(0, _GP)], ins[i], sis[i]
            )

        def out_copy(r, i):
            return pltpu.make_async_copy(
                outs[i],
                out_hbm.at[b, pl.ds(r * _G * _NA, _G * _NA), pl.ds(0, _AP)],
                sos[i],
            )

        in_copy(0, 0).start()
        in_copy(1, 1).start()

        @pl.loop(0, _G, step=2)
        def _rows(r):
            for i in range(2):
                rr = r + i
                in_copy(rr, i).wait()

                @pl.when(rr >= 2)
                def _():
                    out_copy(rr - 2, i).wait()

                _transform_row(ins[i], outs[i], rr)
                out_copy(rr, i).start()

                @pl.when(rr + 2 < _G)
                def _():
                    in_copy(rr + 2, i).start()

        out_copy(_G - 2, 0).wait()
        out_copy(_G - 1, 1).wait()

    return yolo_sc


_yolo_sc = _make_sc_kernel()


def _pad_body(x_ref, o_ref):
    o_ref[:, pl.ds(0, _G), pl.ds(0, _G)] = x_ref[...]


_pad_tc = pl.pallas_call(
    _pad_body,
    grid=(_B, 17),
    in_specs=[
        pl.BlockSpec((None, _CH // 17, _G, _G), lambda b, c: (b, c, 0, 0))
    ],
    out_specs=pl.BlockSpec(
        (None, _CH // 17, _GP, 128), lambda b, c: (b, c, 0, 0)
    ),
    out_shape=jax.ShapeDtypeStruct((_B, _CH, _GP, 128), jnp.float32),
)


@jax.jit
def kernel(x):
    x_p = _pad_tc(x)
    y_p = _yolo_sc(x_p)
    return y_p[:, :, :_ATTRS]


# speedup vs baseline: 1.6302x; 1.0008x over previous
import functools

import jax
import jax.numpy as jnp
from jax import lax
from jax.experimental import pallas as pl
from jax.experimental.pallas import tpu as pltpu
from jax.experimental.pallas import tpu_sc as plsc

_ANCHORS = ((10.0, 13.0), (16.0, 30.0), (33.0, 23.0))
_G = 76
_CELLS = _G * _G
_NA = 3
_ATTRS = 85
_CH = _NA * _ATTRS
_B = 32
_STRIDE = 8.0
_LANES = 16
_WSTARTS = (0, 16, 32, 48, 64)
_GP = 80
_AP = 88


def _transform_row(in_v, out_v, r):
    iota = lax.iota(jnp.int32, _LANES)

    in_mask = iota < (_G - _WSTARTS[-1])

    for a in range(_NA):
        @plsc.parallel_loop(0, _ATTRS, unroll=5)
        def _sigmoid_all(t):
            tb = jnp.zeros((_LANES,), jnp.int32) + t
            for w0 in _WSTARTS:
                rows = (iota + w0) * _NA + a
                v = in_v[a * _ATTRS + t, pl.ds(w0, _LANES)]
                s = 1.0 / (1.0 + jnp.exp(-v))
                m = in_mask if w0 == _WSTARTS[-1] else None
                plsc.store_scatter(out_v, [rows, tb], s, mask=m)

    cy8 = r.astype(jnp.float32) * _STRIDE
    zeros = jnp.zeros((_LANES,), jnp.int32)
    for w0 in _WSTARTS:
        cx8 = (iota + w0).astype(jnp.float32) * _STRIDE
        m = in_mask if w0 == _WSTARTS[-1] else None
        for a in range(_NA):
            base = a * _ATTRS
            rows = (iota + w0) * _NA + a
            vx = in_v[base + 0, pl.ds(w0, _LANES)]
            sx = 1.0 / (1.0 + jnp.exp(-vx))
            plsc.store_scatter(out_v, [rows, zeros], sx * _STRIDE + cx8,
                               mask=m)
            vy = in_v[base + 1, pl.ds(w0, _LANES)]
            sy = 1.0 / (1.0 + jnp.exp(-vy))
            plsc.store_scatter(out_v, [rows, zeros + 1], sy * _STRIDE + cy8,
                               mask=m)
            vw = in_v[base + 2, pl.ds(w0, _LANES)]
            plsc.store_scatter(out_v, [rows, zeros + 2],
                               jnp.exp(vw) * _ANCHORS[a][0], mask=m)
            vh = in_v[base + 3, pl.ds(w0, _LANES)]
            plsc.store_scatter(out_v, [rows, zeros + 3],
                               jnp.exp(vh) * _ANCHORS[a][1], mask=m)


def _make_sc_kernel():
    mesh = plsc.VectorSubcoreMesh(core_axis_name="c", subcore_axis_name="s")

    @functools.partial(
        pl.kernel,
        out_type=jax.ShapeDtypeStruct((_B, _CELLS * _NA, 128), jnp.float32),
        mesh=mesh,
        scratch_types=[
            pltpu.VMEM((_CH, _GP), jnp.float32),
            pltpu.VMEM((_CH, _GP), jnp.float32),
            pltpu.VMEM((_G * _NA, _AP), jnp.float32),
            pltpu.VMEM((_G * _NA, _AP), jnp.float32),
            pltpu.SemaphoreType.DMA,
            pltpu.SemaphoreType.DMA,
            pltpu.SemaphoreType.DMA,
            pltpu.SemaphoreType.DMA,
        ],
        compiler_params=pltpu.CompilerParams(
            use_tc_tiling_on_sc=False, needs_layout_passes=False
        ),
    )
    def yolo_sc(x_hbm, out_hbm, in0, in1, o0, o1, si0, si1, so0, so1):
        b = lax.axis_index("s") * 2 + lax.axis_index("c")
        ins = (in0, in1)
        outs = (o0, o1)
        sis = (si0, si1)
        sos = (so0, so1)
        def in_copy(r, i):
            return pltpu.make_async_copy(
                x_hbm.at[b, :, r, pl.ds(0, _GP)], ins[i], sis[i]
            )

        def out_copy(r, i):
            return pltpu.make_async_copy(
                outs[i],
                out_hbm.at[b, pl.ds(r * _G * _NA, _G * _NA), pl.ds(0, _AP)],
                sos[i],
            )

        in_copy(0, 0).start()
        in_copy(1, 1).start()

        @pl.loop(0, _G, step=2)
        def _rows(r):
            for i in range(2):
                rr = r + i
                in_copy(rr, i).wait()

                @pl.when(rr >= 2)
                def _():
                    out_copy(rr - 2, i).wait()

                _transform_row(ins[i], outs[i], rr)
                out_copy(rr, i).start()

                @pl.when(rr + 2 < _G)
                def _():
                    in_copy(rr + 2, i).start()

        out_copy(_G - 2, 0).wait()
        out_copy(_G - 1, 1).wait()

    return yolo_sc


_yolo_sc = _make_sc_kernel()


def _pad_body(x_ref, o_ref):
    o_ref[...] = jnp.pad(
        x_ref[...], ((0, 0), (0, _GP - _G), (0, 128 - _G))
    )


_pad_tc = pl.pallas_call(
    _pad_body,
    grid=(_B, 17),
    in_specs=[
        pl.BlockSpec((None, _CH // 17, _G, _G), lambda b, c: (b, c, 0, 0))
    ],
    out_specs=pl.BlockSpec(
        (None, _CH // 17, _GP, 128), lambda b, c: (b, c, 0, 0)
    ),
    out_shape=jax.ShapeDtypeStruct((_B, _CH, _GP, 128), jnp.float32),
)


@jax.jit
def kernel(x):
    x_p = _pad_tc(x)
    y_p = _yolo_sc(x_p)
    return y_p[:, :, :_ATTRS]


# speedup vs baseline: 1.9673x; 1.2067x over previous
import functools

import jax
import jax.numpy as jnp
from jax import lax
from jax.experimental import pallas as pl
from jax.experimental.pallas import tpu as pltpu
from jax.experimental.pallas import tpu_sc as plsc

_ANCHORS = ((10.0, 13.0), (16.0, 30.0), (33.0, 23.0))
_G = 76
_CELLS = _G * _G
_NA = 3
_ATTRS = 85
_CH = _NA * _ATTRS
_B = 32
_STRIDE = 8.0
_LANES = 16
_WSTARTS = (0, 16, 32, 48, 64)
_GP = 80
_AP = 88


def _transform_row(in_v, out_v, r):
    iota = lax.iota(jnp.int32, _LANES)

    in_mask = iota < (_G - _WSTARTS[-1])

    for a in range(_NA):
        @plsc.parallel_loop(0, _ATTRS, unroll=5)
        def _sigmoid_all(t):
            tb = jnp.zeros((_LANES,), jnp.int32) + t
            for w0 in _WSTARTS:
                rows = (iota + w0) * _NA + a
                v = in_v[a * _ATTRS + t, pl.ds(w0, _LANES)]
                s = 1.0 / (1.0 + jnp.exp(-v))
                m = in_mask if w0 == _WSTARTS[-1] else None
                plsc.store_scatter(out_v, [rows, tb], s, mask=m)

    cy8 = r.astype(jnp.float32) * _STRIDE
    zeros = jnp.zeros((_LANES,), jnp.int32)
    for w0 in _WSTARTS:
        cx8 = (iota + w0).astype(jnp.float32) * _STRIDE
        m = in_mask if w0 == _WSTARTS[-1] else None
        for a in range(_NA):
            base = a * _ATTRS
            rows = (iota + w0) * _NA + a
            vx = in_v[base + 0, pl.ds(w0, _LANES)]
            sx = 1.0 / (1.0 + jnp.exp(-vx))
            plsc.store_scatter(out_v, [rows, zeros], sx * _STRIDE + cx8,
                               mask=m)
            vy = in_v[base + 1, pl.ds(w0, _LANES)]
            sy = 1.0 / (1.0 + jnp.exp(-vy))
            plsc.store_scatter(out_v, [rows, zeros + 1], sy * _STRIDE + cy8,
                               mask=m)
            vw = in_v[base + 2, pl.ds(w0, _LANES)]
            plsc.store_scatter(out_v, [rows, zeros + 2],
                               jnp.exp(vw) * _ANCHORS[a][0], mask=m)
            vh = in_v[base + 3, pl.ds(w0, _LANES)]
            plsc.store_scatter(out_v, [rows, zeros + 3],
                               jnp.exp(vh) * _ANCHORS[a][1], mask=m)


def _make_sc_kernel():
    mesh = plsc.VectorSubcoreMesh(core_axis_name="c", subcore_axis_name="s")

    @functools.partial(
        pl.kernel,
        out_type=jax.ShapeDtypeStruct((_B, _CELLS * _NA, 128), jnp.float32),
        mesh=mesh,
        scratch_types=[
            pltpu.VMEM((_CH, _GP), jnp.float32),
            pltpu.VMEM((_CH, _GP), jnp.float32),
            pltpu.VMEM((_G * _NA, _AP), jnp.float32),
            pltpu.VMEM((_G * _NA, _AP), jnp.float32),
            pltpu.SemaphoreType.DMA,
            pltpu.SemaphoreType.DMA,
            pltpu.SemaphoreType.DMA,
            pltpu.SemaphoreType.DMA,
        ],
        compiler_params=pltpu.CompilerParams(
            use_tc_tiling_on_sc=False, needs_layout_passes=False
        ),
    )
    def yolo_sc(x_hbm, out_hbm, in0, in1, o0, o1, si0, si1, so0, so1):
        b = lax.axis_index("s") * 2 + lax.axis_index("c")
        ins = (in0, in1)
        outs = (o0, o1)
        sis = (si0, si1)
        sos = (so0, so1)
        def in_copy(r, i):
            return pltpu.make_async_copy(
                x_hbm.at[b, :, r, pl.ds(0, _GP)], ins[i], sis[i]
            )

        def out_copy(r, i):
            return pltpu.make_async_copy(
                outs[i],
                out_hbm.at[b, pl.ds(r * _G * _NA, _G * _NA), pl.ds(0, _AP)],
                sos[i],
            )

        in_copy(0, 0).start()
        in_copy(1, 1).start()

        @pl.loop(0, _G, step=2)
        def _rows(r):
            for i in range(2):
                rr = r + i
                in_copy(rr, i).wait()

                @pl.when(rr >= 2)
                def _():
                    out_copy(rr - 2, i).wait()

                _transform_row(ins[i], outs[i], rr)
                out_copy(rr, i).start()

                @pl.when(rr + 2 < _G)
                def _():
                    in_copy(rr + 2, i).start()

        out_copy(_G - 2, 0).wait()
        out_copy(_G - 1, 1).wait()

    return yolo_sc


_yolo_sc = _make_sc_kernel()


def _pad_body(x_ref, o_ref):
    o_ref[...] = jnp.pad(
        x_ref[...], ((0, 0), (0, _GP - _G), (0, 128 - _G))
    )


_pad_tc = pl.pallas_call(
    _pad_body,
    grid=(_B * 5,),
    in_specs=[
        pl.BlockSpec((None, _CH // 5, _G, _G), lambda i: (i // 5, i % 5, 0, 0))
    ],
    out_specs=pl.BlockSpec(
        (None, _CH // 5, _GP, 128), lambda i: (i // 5, i % 5, 0, 0)
    ),
    out_shape=jax.ShapeDtypeStruct((_B, _CH, _GP, 128), jnp.float32),
)


@jax.jit
def kernel(x):
    x_p = _pad_tc(x)
    y_p = _yolo_sc(x_p)
    return y_p[:, :, :_ATTRS]
